# baseline (speedup 1.0000x reference)
import functools

import jax
import jax.numpy as jnp
from jax import lax
from jax.experimental import pallas as pl
from jax.experimental.pallas import tpu as pltpu
from jax.experimental.pallas import tpu_sc as plsc

E = 8
TOPK = 2
D = 1024
I = 2048
T = 4096
BLK = 256
CORE_CAP = TOPK * T + E * BLK
NB_CORE = CORE_CAP // BLK
NB_SHARED = T // BLK
NB_TOTAL = NB_CORE + NB_SHARED
C_TOTAL = CORE_CAP + T
RCH = 512
SPN = 64


RCH = 512


def _rb_body(x_ref, w16_ref, gv_ref, topw_ref, wsh_ref, slots_ref, sp_ref,
             topi_s):
    w = w16_ref[...]

    def pass1(c, carry):
        xb = x_ref[pl.ds(c * RCH, RCH), :]
        logits = lax.dot_general(xb, w, (((1,), (1,)), ((), ())),
                                 preferred_element_type=jnp.float32)
        l8 = logits[:, :E]
        gv_ref[pl.ds(c * RCH, RCH), :] = l8
        m = jnp.max(l8, axis=1, keepdims=True)
        p = jnp.exp(l8 - m)
        p = p / jnp.sum(p, axis=1, keepdims=True)
        lane = lax.broadcasted_iota(jnp.int32, p.shape, 1)
        w1 = jnp.max(p, axis=1, keepdims=True)
        i1 = jnp.min(jnp.where(p >= w1, lane, E), axis=1, keepdims=True)
        p2 = jnp.where(lane == i1, -1.0, p)
        w2 = jnp.max(p2, axis=1, keepdims=True)
        i2 = jnp.min(jnp.where(p2 >= w2, lane, E), axis=1, keepdims=True)
        topw_ref[pl.ds(c * RCH, RCH), :] = jnp.concatenate([w1, w2], axis=1)
        wsh_ref[pl.ds(c * RCH, RCH), :] = jax.nn.sigmoid(logits[:, E:E + 1])
        ti = jnp.concatenate([i1, i2], axis=1)
        topi_s[pl.ds(c * RCH, RCH), :] = ti
        lanec = lax.broadcasted_iota(jnp.int32, (RCH, E), 1)
        ohc = ((ti[:, 0:1] == lanec).astype(jnp.float32)
               + (ti[:, 1:2] == lanec).astype(jnp.float32))
        return carry + jnp.sum(ohc, axis=0, keepdims=True)

    counts = lax.fori_loop(0, T // RCH, pass1, jnp.zeros((1, E), jnp.float32))

    pc = jnp.ceil(counts / BLK) * BLK
    er = lax.broadcasted_iota(jnp.int32, (E, E), 0)
    ec = lax.broadcasted_iota(jnp.int32, (E, E), 1)
    upper = (er < ec).astype(jnp.float32)
    poff = lax.dot_general(pc, upper, (((1,), (0,)), ((), ())),
                           preferred_element_type=jnp.float32)
    pcum = poff + pc

    rr = lax.broadcasted_iota(jnp.int32, (RCH, RCH), 0)
    rc = lax.broadcasted_iota(jnp.int32, (RCH, RCH), 1)
    lstrict = (rr > rc).astype(jnp.float32)

    def pass2(c, carry):
        tic = topi_s[pl.ds(c * RCH, RCH), :]
        lanec = lax.broadcasted_iota(jnp.int32, (RCH, E), 1)
        oh0 = (tic[:, 0:1] == lanec).astype(jnp.float32)
        oh1 = (tic[:, 1:2] == lanec).astype(jnp.float32)
        ohc = oh0 + oh1
        cume = carry + lax.dot_general(
            lstrict, ohc, (((1,), (0,)), ((), ())),
            preferred_element_type=jnp.float32)
        slotf = poff + cume
        s0 = jnp.sum(oh0 * slotf, axis=1, keepdims=True)
        s1 = jnp.sum(oh1 * slotf, axis=1, keepdims=True)
        slots_ref[pl.ds(c * RCH, RCH), :] = jnp.concatenate(
            [s0, s1], axis=1).astype(jnp.int32)
        return carry + jnp.sum(ohc, axis=0, keepdims=True)

    lax.fori_loop(0, T // RCH, pass2, jnp.zeros((1, E), jnp.float32))

    biota = lax.broadcasted_iota(jnp.int32, (8, SPN), 1)
    bf = (biota * BLK).astype(jnp.float32)
    bex = jnp.zeros((8, SPN), jnp.int32)
    for e in range(E - 1):
        bex = bex + (bf >= pcum[0:1, e:e + 1]).astype(jnp.int32)
    active = (bf < pcum[0:1, E - 1:E]).astype(jnp.int32)
    sp_ref[:, 0:SPN] = bex
    sp_ref[:, SPN:2 * SPN] = active


def _router_binning(x, w16):
    return pl.pallas_call(
        _rb_body,
        out_shape=[
            jax.ShapeDtypeStruct((T, E), jnp.float32),
            jax.ShapeDtypeStruct((T, TOPK), jnp.float32),
            jax.ShapeDtypeStruct((T, 1), jnp.float32),
            jax.ShapeDtypeStruct((T, TOPK), jnp.int32),
            jax.ShapeDtypeStruct((8, 2 * SPN), jnp.int32),
        ],
        scratch_shapes=[pltpu.VMEM((T, TOPK), jnp.int32)],
    )(x, w16)


def _sc_mesh():
    return plsc.VectorSubcoreMesh(core_axis_name="c", subcore_axis_name="s")


_NC = 2
_NW = 32
_TPW = T // _NW
_CHT = 32
_NCHT = _TPW // _CHT


def _g1_body(x_hbm, s0_hbm, s1_hbm, xg_hbm,
             rows_a, rows_b, idx0_a, idx0_b, idx1_a, idx1_b,
             lsem_a, lsem_b, sem0, sem1):
    wid = lax.axis_index("s") * _NC + lax.axis_index("c")
    base = wid * _TPW
    rows = [rows_a, rows_b]
    idx0 = [idx0_a, idx0_b]
    idx1 = [idx1_a, idx1_b]
    lsem = [lsem_a, lsem_b]

    def load(i, b):
        tb = base + i * _CHT
        pltpu.make_async_copy(x_hbm.at[pl.ds(tb, _CHT)], rows[b], lsem[b]).start()
        pltpu.make_async_copy(s0_hbm.at[pl.ds(tb, _CHT)], idx0[b], lsem[b]).start()
        pltpu.make_async_copy(s1_hbm.at[pl.ds(tb, _CHT)], idx1[b], lsem[b]).start()

    def wait_load(b):
        pltpu.make_async_copy(x_hbm.at[pl.ds(base, _CHT)], rows[b], lsem[b]).wait()
        pltpu.make_async_copy(s0_hbm.at[pl.ds(base, _CHT)], idx0[b], lsem[b]).wait()
        pltpu.make_async_copy(s1_hbm.at[pl.ds(base, _CHT)], idx1[b], lsem[b]).wait()

    load(0, 0)
    pend = [None, None]
    for i in range(_NCHT):
        b = i % 2
        if i + 1 < _NCHT:
            nb = 1 - b
            if pend[nb] is not None:
                pend[nb][0].wait()
                pend[nb][1].wait()
                pend[nb] = None
        wait_load(b)
        if i + 1 < _NCHT:
            load(i + 1, 1 - b)
        c0 = pltpu.make_async_copy(rows[b], xg_hbm.at[idx0[b]], sem0)
        c1 = pltpu.make_async_copy(rows[b], xg_hbm.at[idx1[b]], sem1)
        c0.start()
        c1.start()
        pend[b] = (c0, c1)
    for b in range(2):
        if pend[b] is not None:
            pend[b][0].wait()
            pend[b][1].wait()


def _g1(x, slot0, slot1):
    return pl.kernel(
        _g1_body,
        out_type=jax.ShapeDtypeStruct((CORE_CAP, D), jnp.float32),
        mesh=_sc_mesh(),
        scratch_types=[
            pltpu.VMEM((_CHT, D), jnp.float32),
            pltpu.VMEM((_CHT, D), jnp.float32),
            pltpu.VMEM((_CHT,), jnp.int32),
            pltpu.VMEM((_CHT,), jnp.int32),
            pltpu.VMEM((_CHT,), jnp.int32),
            pltpu.VMEM((_CHT,), jnp.int32),
            pltpu.SemaphoreType.DMA,
            pltpu.SemaphoreType.DMA,
            pltpu.SemaphoreType.DMA,
            pltpu.SemaphoreType.DMA,
        ],
    )(x, slot0, slot1)


def _expert_body(sp_ref, xg_ref, wgp_ref, wup_ref, wdn_ref, y_ref):
    i = pl.program_id(0)

    @pl.when(sp_ref[SPN + i] == 1)
    def _():
        xb = xg_ref[...]
        g = lax.dot_general(xb, wgp_ref[0], (((1,), (1,)), ((), ())),
                            preferred_element_type=jnp.float32)
        u = lax.dot_general(xb, wup_ref[0], (((1,), (1,)), ((), ())),
                            preferred_element_type=jnp.float32)
        h = (g * jax.nn.sigmoid(g)) * u
        y_ref[...] = lax.dot_general(h, wdn_ref[0], (((1,), (1,)), ((), ())),
                                     preferred_element_type=jnp.float32)


def _experts(sp, xg, wgp, wup, wdn):
    grid_spec = pltpu.PrefetchScalarGridSpec(
        num_scalar_prefetch=1,
        grid=(NB_CORE,),
        in_specs=[
            pl.BlockSpec((BLK, D), lambda i, sp: (i, 0)),
            pl.BlockSpec((1, I, D), lambda i, sp: (sp[i], 0, 0)),
            pl.BlockSpec((1, I, D), lambda i, sp: (sp[i], 0, 0)),
            pl.BlockSpec((1, D, I), lambda i, sp: (sp[i], 0, 0)),
        ],
        out_specs=pl.BlockSpec((BLK, D), lambda i, sp: (i, 0)),
    )
    return pl.pallas_call(
        _expert_body,
        grid_spec=grid_spec,
        out_shape=jax.ShapeDtypeStruct((CORE_CAP, D), jnp.float32),
        compiler_params=pltpu.CompilerParams(
            dimension_semantics=("arbitrary",)),
    )(sp, xg, wgp, wup, wdn)


def _g2_body(y_hbm, s0_hbm, s1_hbm, y1_hbm, y2_hbm,
             rows0_v, rows1_v, idx0_v, idx1_v, sem0, sem1):
    wid = lax.axis_index("s") * _NC + lax.axis_index("c")
    base = wid * _TPW

    def chunk(i, carry):
        tb = base + i * _CHT
        pltpu.sync_copy(s0_hbm.at[pl.ds(tb, _CHT)], idx0_v)
        pltpu.sync_copy(s1_hbm.at[pl.ds(tb, _CHT)], idx1_v)
        c0 = pltpu.make_async_copy(y_hbm.at[idx0_v], rows0_v, sem0)
        c1 = pltpu.make_async_copy(y_hbm.at[idx1_v], rows1_v, sem1)
        c0.start()
        c1.start()
        c0.wait()
        c1.wait()
        pltpu.sync_copy(rows0_v, y1_hbm.at[pl.ds(tb, _CHT)])
        pltpu.sync_copy(rows1_v, y2_hbm.at[pl.ds(tb, _CHT)])
        return carry

    lax.fori_loop(0, _NCHT, chunk, 0)


def _g2(y, slot0, slot1):
    return pl.kernel(
        _g2_body,
        out_type=[
            jax.ShapeDtypeStruct((T, D), jnp.float32),
            jax.ShapeDtypeStruct((T, D), jnp.float32),
        ],
        mesh=_sc_mesh(),
        scratch_types=[
            pltpu.VMEM((_CHT, D), jnp.float32),
            pltpu.VMEM((_CHT, D), jnp.float32),
            pltpu.VMEM((_CHT,), jnp.int32),
            pltpu.VMEM((_CHT,), jnp.int32),
            pltpu.SemaphoreType.DMA,
            pltpu.SemaphoreType.DMA,
        ],
    )(y, slot0, slot1)


def _shared_body(x_ref, wsg_ref, wsu_ref, wsd_ref, ysh_ref):
    xb = x_ref[...]
    g = lax.dot_general(xb, wsg_ref[...], (((1,), (1,)), ((), ())),
                        preferred_element_type=jnp.float32)
    u = lax.dot_general(xb, wsu_ref[...], (((1,), (1,)), ((), ())),
                        preferred_element_type=jnp.float32)
    h = (g * jax.nn.sigmoid(g)) * u
    ysh_ref[...] = lax.dot_general(h, wsd_ref[...], (((1,), (1,)), ((), ())),
                                   preferred_element_type=jnp.float32)


def _shared(x, wsg, wsu, wsd):
    SB = 256
    return pl.pallas_call(
        _shared_body,
        grid=(T // SB,),
        in_specs=[
            pl.BlockSpec((SB, D), lambda i: (i, 0)),
            pl.BlockSpec((I, D), lambda i: (0, 0)),
            pl.BlockSpec((I, D), lambda i: (0, 0)),
            pl.BlockSpec((D, I), lambda i: (0, 0)),
        ],
        out_specs=pl.BlockSpec((SB, D), lambda i: (i, 0)),
        out_shape=jax.ShapeDtypeStruct((T, D), jnp.float32),
        compiler_params=pltpu.CompilerParams(
            dimension_semantics=("arbitrary",)),
    )(x, wsg, wsu, wsd)


def _combine_body(y1_ref, y2_ref, y3_ref, tw_ref, wsh_ref, o_ref):
    tw = tw_ref[...]
    o_ref[...] = (tw[:, 0:1] * y1_ref[...]
                  + tw[:, 1:2] * y2_ref[...]
                  + wsh_ref[...] * y3_ref[...])


def _combine(y1g, y2g, ysh, topw, wsh):
    CBB = 512
    return pl.pallas_call(
        _combine_body,
        grid=(T // CBB,),
        in_specs=[
            pl.BlockSpec((CBB, D), lambda i: (i, 0)),
            pl.BlockSpec((CBB, D), lambda i: (i, 0)),
            pl.BlockSpec((CBB, D), lambda i: (i, 0)),
            pl.BlockSpec((CBB, TOPK), lambda i: (i, 0)),
            pl.BlockSpec((CBB, 1), lambda i: (i, 0)),
        ],
        out_specs=pl.BlockSpec((CBB, D), lambda i: (i, 0)),
        out_shape=jax.ShapeDtypeStruct((T, D), jnp.float32),
        compiler_params=pltpu.CompilerParams(
            dimension_semantics=("arbitrary",)),
    )(y1g, y2g, ysh, topw, wsh)


def kernel(hidden_states, Wg, Wgp, Wup, Wdn, Wsg, Wsu, Wsd, Wshg):
    Bq, Sq, Dq = hidden_states.shape
    x = hidden_states.reshape(-1, Dq)
    w16 = jnp.concatenate(
        [Wg, Wshg, jnp.zeros((16 - E - 1, Dq), jnp.float32)], axis=0)
    gate_vals, topw, wsh, slots, sp = _router_binning(x, w16)
    slot0 = slots[:, 0]
    slot1 = slots[:, 1]
    spv = sp[0]
    xg = _g1(x, slot0, slot1)
    ysh = _shared(x, Wsg, Wsu, Wsd)
    y = _experts(spv, xg, Wgp, Wup, Wdn)
    y1g, y2g = _g2(y, slot0, slot1)
    out = _combine(y1g, y2g, ysh, topw, wsh)
    return out.reshape(Bq, Sq, Dq), gate_vals


# speedup vs baseline: 1.0245x; 1.0245x over previous
import functools

import jax
import jax.numpy as jnp
from jax import lax
from jax.experimental import pallas as pl
from jax.experimental.pallas import tpu as pltpu
from jax.experimental.pallas import tpu_sc as plsc

E = 8
TOPK = 2
D = 1024
I = 2048
T = 4096
BLK = 256
CORE_CAP = TOPK * T + E * BLK
NB_CORE = CORE_CAP // BLK
NB_SHARED = T // BLK
NB_TOTAL = NB_CORE + NB_SHARED
C_TOTAL = CORE_CAP + T
RCH = 512
SPN = 64


def _router_body(x_ref, w16_ref, gv_ref, topw_ref, topi_ref, wsh_ref):
    x = x_ref[...]
    w = w16_ref[...]
    logits = lax.dot_general(x, w, (((1,), (1,)), ((), ())),
                             preferred_element_type=jnp.float32)
    l8 = logits[:, :E]
    gv_ref[...] = l8
    m = jnp.max(l8, axis=1, keepdims=True)
    p = jnp.exp(l8 - m)
    p = p / jnp.sum(p, axis=1, keepdims=True)
    lane = lax.broadcasted_iota(jnp.int32, p.shape, 1)
    w1 = jnp.max(p, axis=1, keepdims=True)
    i1 = jnp.min(jnp.where(p >= w1, lane, E), axis=1, keepdims=True)
    p2 = jnp.where(lane == i1, -1.0, p)
    w2 = jnp.max(p2, axis=1, keepdims=True)
    i2 = jnp.min(jnp.where(p2 >= w2, lane, E), axis=1, keepdims=True)
    topw_ref[...] = jnp.concatenate([w1, w2], axis=1)
    topi_ref[...] = jnp.concatenate([i1, i2], axis=1)
    wsh_ref[...] = jax.nn.sigmoid(logits[:, E:E + 1])


def _router(x, w16):
    RB = 512
    grid = (T // RB,)
    return pl.pallas_call(
        _router_body,
        grid=grid,
        in_specs=[
            pl.BlockSpec((RB, D), lambda i: (i, 0)),
            pl.BlockSpec((16, D), lambda i: (0, 0)),
        ],
        out_specs=[
            pl.BlockSpec((RB, E), lambda i: (i, 0)),
            pl.BlockSpec((RB, TOPK), lambda i: (i, 0)),
            pl.BlockSpec((RB, TOPK), lambda i: (i, 0)),
            pl.BlockSpec((RB, 1), lambda i: (i, 0)),
        ],
        out_shape=[
            jax.ShapeDtypeStruct((T, E), jnp.float32),
            jax.ShapeDtypeStruct((T, TOPK), jnp.float32),
            jax.ShapeDtypeStruct((T, TOPK), jnp.int32),
            jax.ShapeDtypeStruct((T, 1), jnp.float32),
        ],
    )(x, w16)


def _binning_body(topi_ref, slots_ref, sp_ref):
    ti = topi_ref[...]
    lane8 = lax.broadcasted_iota(jnp.int32, (T, E), 1)
    oh = ((ti[:, 0:1] == lane8).astype(jnp.float32)
          + (ti[:, 1:2] == lane8).astype(jnp.float32))
    counts = jnp.sum(oh, axis=0, keepdims=True)
    pc = jnp.ceil(counts / BLK) * BLK
    er = lax.broadcasted_iota(jnp.int32, (E, E), 0)
    ec = lax.broadcasted_iota(jnp.int32, (E, E), 1)
    upper = (er < ec).astype(jnp.float32)
    poff = lax.dot_general(pc, upper, (((1,), (0,)), ((), ())),
                           preferred_element_type=jnp.float32)
    pcum = poff + pc

    rr = lax.broadcasted_iota(jnp.int32, (RCH, RCH), 0)
    rc = lax.broadcasted_iota(jnp.int32, (RCH, RCH), 1)
    lstrict = (rr > rc).astype(jnp.float32)

    def step(c, carry):
        tic = topi_ref[pl.ds(c * RCH, RCH), :]
        lanec = lax.broadcasted_iota(jnp.int32, (RCH, E), 1)
        oh0 = (tic[:, 0:1] == lanec).astype(jnp.float32)
        oh1 = (tic[:, 1:2] == lanec).astype(jnp.float32)
        ohc = oh0 + oh1
        cume = carry + lax.dot_general(
            lstrict, ohc, (((1,), (0,)), ((), ())),
            preferred_element_type=jnp.float32)
        slotf = poff + cume
        s0 = jnp.sum(oh0 * slotf, axis=1, keepdims=True)
        s1 = jnp.sum(oh1 * slotf, axis=1, keepdims=True)
        slots_ref[pl.ds(c * RCH, RCH), :] = jnp.concatenate(
            [s0, s1], axis=1).astype(jnp.int32)
        return carry + jnp.sum(ohc, axis=0, keepdims=True)

    lax.fori_loop(0, T // RCH, step, jnp.zeros((1, E), jnp.float32))

    biota = lax.broadcasted_iota(jnp.int32, (8, SPN), 1)
    bf = (biota * BLK).astype(jnp.float32)
    bex = jnp.zeros((8, SPN), jnp.int32)
    for e in range(E - 1):
        bex = bex + (bf >= pcum[0:1, e:e + 1]).astype(jnp.int32)
    active = (bf < pcum[0:1, E - 1:E]).astype(jnp.int32)
    sp_ref[:, 0:SPN] = bex
    sp_ref[:, SPN:2 * SPN] = active


def _binning(topi):
    return pl.pallas_call(
        _binning_body,
        out_shape=[
            jax.ShapeDtypeStruct((T, TOPK), jnp.int32),
            jax.ShapeDtypeStruct((8, 2 * SPN), jnp.int32),
        ],
    )(topi)


def _sc_mesh():
    return plsc.VectorSubcoreMesh(core_axis_name="c", subcore_axis_name="s")


_NC = 2
_NW = 32
_TPW = T // _NW
_CHT = 32
_NCHT = _TPW // _CHT


def _g1_body(x_hbm, s0_hbm, s1_hbm, xg_hbm,
             rows_a, rows_b, idx0_a, idx0_b, idx1_a, idx1_b,
             lsem_a, lsem_b, sem0, sem1):
    wid = lax.axis_index("s") * _NC + lax.axis_index("c")
    base = wid * _TPW
    rows = [rows_a, rows_b]
    idx0 = [idx0_a, idx0_b]
    idx1 = [idx1_a, idx1_b]
    lsem = [lsem_a, lsem_b]

    def load(i, b):
        tb = base + i * _CHT
        pltpu.make_async_copy(x_hbm.at[pl.ds(tb, _CHT)], rows[b], lsem[b]).start()
        pltpu.make_async_copy(s0_hbm.at[pl.ds(tb, _CHT)], idx0[b], lsem[b]).start()
        pltpu.make_async_copy(s1_hbm.at[pl.ds(tb, _CHT)], idx1[b], lsem[b]).start()

    def wait_load(b):
        pltpu.make_async_copy(x_hbm.at[pl.ds(base, _CHT)], rows[b], lsem[b]).wait()
        pltpu.make_async_copy(s0_hbm.at[pl.ds(base, _CHT)], idx0[b], lsem[b]).wait()
        pltpu.make_async_copy(s1_hbm.at[pl.ds(base, _CHT)], idx1[b], lsem[b]).wait()

    load(0, 0)
    pend = [None, None]
    for i in range(_NCHT):
        b = i % 2
        if i + 1 < _NCHT:
            nb = 1 - b
            if pend[nb] is not None:
                pend[nb][0].wait()
                pend[nb][1].wait()
                pend[nb] = None
        wait_load(b)
        if i + 1 < _NCHT:
            load(i + 1, 1 - b)
        c0 = pltpu.make_async_copy(rows[b], xg_hbm.at[idx0[b]], sem0)
        c1 = pltpu.make_async_copy(rows[b], xg_hbm.at[idx1[b]], sem1)
        c0.start()
        c1.start()
        pend[b] = (c0, c1)
    for b in range(2):
        if pend[b] is not None:
            pend[b][0].wait()
            pend[b][1].wait()


def _g1(x, slot0, slot1):
    return pl.kernel(
        _g1_body,
        out_type=jax.ShapeDtypeStruct((CORE_CAP, D), jnp.float32),
        mesh=_sc_mesh(),
        scratch_types=[
            pltpu.VMEM((_CHT, D), jnp.float32),
            pltpu.VMEM((_CHT, D), jnp.float32),
            pltpu.VMEM((_CHT,), jnp.int32),
            pltpu.VMEM((_CHT,), jnp.int32),
            pltpu.VMEM((_CHT,), jnp.int32),
            pltpu.VMEM((_CHT,), jnp.int32),
            pltpu.SemaphoreType.DMA,
            pltpu.SemaphoreType.DMA,
            pltpu.SemaphoreType.DMA,
            pltpu.SemaphoreType.DMA,
        ],
    )(x, slot0, slot1)


def _expert_body(sp_ref, xg_ref, wgp_ref, wup_ref, wdn_ref, y_ref):
    i = pl.program_id(0)

    @pl.when(sp_ref[SPN + i] == 1)
    def _():
        xb = xg_ref[...]
        g = lax.dot_general(xb, wgp_ref[0], (((1,), (1,)), ((), ())),
                            preferred_element_type=jnp.float32)
        u = lax.dot_general(xb, wup_ref[0], (((1,), (1,)), ((), ())),
                            preferred_element_type=jnp.float32)
        h = (g * jax.nn.sigmoid(g)) * u
        y_ref[...] = lax.dot_general(h, wdn_ref[0], (((1,), (1,)), ((), ())),
                                     preferred_element_type=jnp.float32)


def _experts(sp, xg, wgp, wup, wdn):
    grid_spec = pltpu.PrefetchScalarGridSpec(
        num_scalar_prefetch=1,
        grid=(NB_CORE,),
        in_specs=[
            pl.BlockSpec((BLK, D), lambda i, sp: (i, 0)),
            pl.BlockSpec((1, I, D), lambda i, sp: (sp[i], 0, 0)),
            pl.BlockSpec((1, I, D), lambda i, sp: (sp[i], 0, 0)),
            pl.BlockSpec((1, D, I), lambda i, sp: (sp[i], 0, 0)),
        ],
        out_specs=pl.BlockSpec((BLK, D), lambda i, sp: (i, 0)),
    )
    return pl.pallas_call(
        _expert_body,
        grid_spec=grid_spec,
        out_shape=jax.ShapeDtypeStruct((CORE_CAP, D), jnp.float32),
        compiler_params=pltpu.CompilerParams(
            dimension_semantics=("arbitrary",)),
    )(sp, xg, wgp, wup, wdn)


def _g2_body(y_hbm, s0_hbm, s1_hbm, y1_hbm, y2_hbm,
             rows0_v, rows1_v, idx0_v, idx1_v, sem0, sem1):
    wid = lax.axis_index("s") * _NC + lax.axis_index("c")
    base = wid * _TPW

    def chunk(i, carry):
        tb = base + i * _CHT
        pltpu.sync_copy(s0_hbm.at[pl.ds(tb, _CHT)], idx0_v)
        pltpu.sync_copy(s1_hbm.at[pl.ds(tb, _CHT)], idx1_v)
        c0 = pltpu.make_async_copy(y_hbm.at[idx0_v], rows0_v, sem0)
        c1 = pltpu.make_async_copy(y_hbm.at[idx1_v], rows1_v, sem1)
        c0.start()
        c1.start()
        c0.wait()
        c1.wait()
        pltpu.sync_copy(rows0_v, y1_hbm.at[pl.ds(tb, _CHT)])
        pltpu.sync_copy(rows1_v, y2_hbm.at[pl.ds(tb, _CHT)])
        return carry

    lax.fori_loop(0, _NCHT, chunk, 0)


def _g2(y, slot0, slot1):
    return pl.kernel(
        _g2_body,
        out_type=[
            jax.ShapeDtypeStruct((T, D), jnp.float32),
            jax.ShapeDtypeStruct((T, D), jnp.float32),
        ],
        mesh=_sc_mesh(),
        scratch_types=[
            pltpu.VMEM((_CHT, D), jnp.float32),
            pltpu.VMEM((_CHT, D), jnp.float32),
            pltpu.VMEM((_CHT,), jnp.int32),
            pltpu.VMEM((_CHT,), jnp.int32),
            pltpu.SemaphoreType.DMA,
            pltpu.SemaphoreType.DMA,
        ],
    )(y, slot0, slot1)


def _shared_body(x_ref, wsg_ref, wsu_ref, wsd_ref, ysh_ref):
    xb = x_ref[...]
    g = lax.dot_general(xb, wsg_ref[...], (((1,), (1,)), ((), ())),
                        preferred_element_type=jnp.float32)
    u = lax.dot_general(xb, wsu_ref[...], (((1,), (1,)), ((), ())),
                        preferred_element_type=jnp.float32)
    h = (g * jax.nn.sigmoid(g)) * u
    ysh_ref[...] = lax.dot_general(h, wsd_ref[...], (((1,), (1,)), ((), ())),
                                   preferred_element_type=jnp.float32)


def _shared(x, wsg, wsu, wsd):
    SB = 256
    return pl.pallas_call(
        _shared_body,
        grid=(T // SB,),
        in_specs=[
            pl.BlockSpec((SB, D), lambda i: (i, 0)),
            pl.BlockSpec((I, D), lambda i: (0, 0)),
            pl.BlockSpec((I, D), lambda i: (0, 0)),
            pl.BlockSpec((D, I), lambda i: (0, 0)),
        ],
        out_specs=pl.BlockSpec((SB, D), lambda i: (i, 0)),
        out_shape=jax.ShapeDtypeStruct((T, D), jnp.float32),
        compiler_params=pltpu.CompilerParams(
            dimension_semantics=("arbitrary",)),
    )(x, wsg, wsu, wsd)


def _combine_body(y1_ref, y2_ref, y3_ref, tw_ref, wsh_ref, o_ref):
    tw = tw_ref[...]
    o_ref[...] = (tw[:, 0:1] * y1_ref[...]
                  + tw[:, 1:2] * y2_ref[...]
                  + wsh_ref[...] * y3_ref[...])


def _combine(y1g, y2g, ysh, topw, wsh):
    CBB = 512
    return pl.pallas_call(
        _combine_body,
        grid=(T // CBB,),
        in_specs=[
            pl.BlockSpec((CBB, D), lambda i: (i, 0)),
            pl.BlockSpec((CBB, D), lambda i: (i, 0)),
            pl.BlockSpec((CBB, D), lambda i: (i, 0)),
            pl.BlockSpec((CBB, TOPK), lambda i: (i, 0)),
            pl.BlockSpec((CBB, 1), lambda i: (i, 0)),
        ],
        out_specs=pl.BlockSpec((CBB, D), lambda i: (i, 0)),
        out_shape=jax.ShapeDtypeStruct((T, D), jnp.float32),
        compiler_params=pltpu.CompilerParams(
            dimension_semantics=("arbitrary",)),
    )(y1g, y2g, ysh, topw, wsh)


def kernel(hidden_states, Wg, Wgp, Wup, Wdn, Wsg, Wsu, Wsd, Wshg):
    Bq, Sq, Dq = hidden_states.shape
    x = hidden_states.reshape(-1, Dq)
    w16 = jnp.concatenate(
        [Wg, Wshg, jnp.zeros((16 - E - 1, Dq), jnp.float32)], axis=0)
    gate_vals, topw, topi, wsh = _router(x, w16)
    slots, sp = _binning(topi)
    slot0 = slots[:, 0]
    slot1 = slots[:, 1]
    spv = sp[0]
    xg = _g1(x, slot0, slot1)
    ysh = _shared(x, Wsg, Wsu, Wsd)
    y = _experts(spv, xg, Wgp, Wup, Wdn)
    y1g, y2g = _g2(y, slot0, slot1)
    out = _combine(y1g, y2g, ysh, topw, wsh)
    return out.reshape(Bq, Sq, Dq), gate_vals
